# Initial kernel scaffold; baseline (speedup 1.0000x reference)
import jax
import jax.numpy as jnp
from jax.experimental import pallas as pl


def kernel(inputs, seq_len, embed_table, W1, b1, W2, b2):
    raise NotImplementedError("write your pallas kernel here")


# speedup vs baseline: 3.2754x; 3.2754x over previous
import jax
import jax.numpy as jnp
from jax.experimental import pallas as pl
from jax.experimental.pallas import tpu as pltpu

_B, _S, _D = 16, 2048, 512
_NV, _H = 32, 512
_T = _S + _NV


def _body(seq_ref, emb_ref, w1_ref, b1_ref, w2_ref, b2_ref, inp_ref,
          out_ref, virt_ref):
    b = pl.program_id(0)

    @pl.when(b == 0)
    def _compute_virtual():
        h = jnp.tanh(
            jnp.dot(emb_ref[...], w1_ref[...],
                    preferred_element_type=jnp.float32) + b1_ref[...])
        virt_ref[...] = jnp.dot(
            h, w2_ref[...], preferred_element_type=jnp.float32) + b2_ref[...]

    seq = seq_ref[b]
    t = jax.lax.broadcasted_iota(jnp.int32, (_S, _D), 0)
    out_ref[0, :_S, :] = jnp.where(t < seq, inp_ref[0], 0.0)
    out_ref[0, _S:, :] = jnp.zeros((_NV, _D), jnp.float32)

    base = pl.multiple_of((seq // 8) * 8, 8)
    r = seq - base
    _W = _NV + 8
    i_w = jax.lax.broadcasted_iota(jnp.int32, (_W, _NV), 0)
    j_w = jax.lax.broadcasted_iota(jnp.int32, (_W, _NV), 1)
    onehot = (i_w - r == j_w).astype(jnp.float32)
    win_virt = jnp.dot(onehot, virt_ref[...],
                       preferred_element_type=jnp.float32)
    existing = out_ref[0, pl.ds(base, _W), :]
    i_col = jax.lax.broadcasted_iota(jnp.int32, (_W, _D), 0)
    out_ref[0, pl.ds(base, _W), :] = jnp.where(i_col < r, existing, win_virt)


def kernel(inputs, seq_len, embed_table, W1, b1, W2, b2):
    seq_len = seq_len.astype(jnp.int32)
    out = pl.pallas_call(
        _body,
        grid=(_B,),
        in_specs=[
            pl.BlockSpec(memory_space=pltpu.SMEM),
            pl.BlockSpec((_NV, _H), lambda b: (0, 0)),
            pl.BlockSpec((_H, _H), lambda b: (0, 0)),
            pl.BlockSpec((1, _H), lambda b: (0, 0)),
            pl.BlockSpec((_H, _D), lambda b: (0, 0)),
            pl.BlockSpec((1, _D), lambda b: (0, 0)),
            pl.BlockSpec((1, _S, _D), lambda b: (b, 0, 0)),
        ],
        out_specs=pl.BlockSpec((1, _T, _D), lambda b: (b, 0, 0)),
        out_shape=jax.ShapeDtypeStruct((_B, _T, _D), jnp.float32),
        scratch_shapes=[pltpu.VMEM((_NV, _D), jnp.float32)],
    )(seq_len, embed_table, W1, b1.reshape(1, _H), W2, b2.reshape(1, _D),
      inputs)
    return out, seq_len + _NV
